# Initial kernel scaffold; baseline (speedup 1.0000x reference)
import jax
import jax.numpy as jnp
from jax.experimental import pallas as pl


def kernel(x, edge_index, emb_table, W_self, W_neigh, lin_W, lin_b):
    raise NotImplementedError("write your pallas kernel here")


# speedup vs baseline: 4.0982x; 4.0982x over previous
import functools

import jax
import jax.numpy as jnp
from jax import lax
from jax.experimental import pallas as pl
from jax.experimental.pallas import tpu as pltpu
from jax.experimental.pallas import tpu_sc as plsc

NC = 2
NS = 16
NW = NC * NS


def _make_embed(N, L, V, D):
  assert L == 16 and D == 128
  nodes_per_batch = 8
  nb_total = N // nodes_per_batch
  assert N % nodes_per_batch == 0
  iters = (nb_total + NW - 1) // NW

  mesh = plsc.VectorSubcoreMesh(core_axis_name="c", subcore_axis_name="s",
                                num_cores=NC, num_subcores=NS)

  @functools.partial(
      pl.kernel, mesh=mesh,
      out_type=jax.ShapeDtypeStruct((N, D), jnp.float32),
      scratch_types=[
          pltpu.VMEM((nodes_per_batch * L,), jnp.int32),
          pltpu.VMEM((nodes_per_batch * L, D), jnp.float32),
          pltpu.VMEM((nodes_per_batch, D), jnp.float32),
          pltpu.SemaphoreType.DMA,
      ],
  )
  def embed(x_hbm, table_hbm, h_hbm, idx_v, rows_v, hbuf, sem):
    wid = lax.axis_index("s") * NC + lax.axis_index("c")

    def body(i, _):
      b = i * NW + wid

      @pl.when(b < nb_total)
      def _():
        pltpu.sync_copy(x_hbm.at[pl.ds(pl.multiple_of(b * nodes_per_batch * L, 8),
                                       nodes_per_batch * L)], idx_v)
        pltpu.async_copy(table_hbm.at[idx_v], rows_v, sem).wait()
        for n in range(nodes_per_batch):
          for c in range(D // 16):
            cols = pl.ds(c * 16, 16)
            vals = [rows_v[n * L + t, cols] for t in range(L)]
            while len(vals) > 1:
              vals = [vals[2 * k] + vals[2 * k + 1]
                      for k in range(len(vals) // 2)] + vals[len(vals) & ~1:]
            acc = vals[0] * (1.0 / L)
            hbuf[n, cols] = jnp.maximum(acc, 0.0)
        pltpu.sync_copy(hbuf, h_hbm.at[pl.ds(pl.multiple_of(b * nodes_per_batch, 8),
                                             nodes_per_batch)])
      return ()

    lax.fori_loop(0, iters, body, (), unroll=False)

  return embed


def _make_agg(N, E, D):
  assert D == 128 and N % NS == 0
  batch = 80
  per_worker = E // NW
  assert E % NW == 0 and per_worker % batch == 0
  iters = per_worker // batch
  rows_a = (N // NS) & ~7
  rem = N - NS * rows_a
  assert rem % 8 == 0

  mesh = plsc.VectorSubcoreMesh(core_axis_name="c", subcore_axis_name="s",
                                num_cores=NC, num_subcores=NS)

  @functools.partial(
      pl.kernel, mesh=mesh,
      out_type=[jax.ShapeDtypeStruct((NC, N, D), jnp.float32),
                jax.ShapeDtypeStruct((NC, N, D), jnp.float32)],
      scratch_types=[
          pltpu.VMEM((batch,), jnp.int32),
          pltpu.VMEM((batch,), jnp.int32),
          pltpu.VMEM((batch, D), jnp.float32),
          pltpu.VMEM((batch, D), jnp.float32),
          pltpu.VMEM_SHARED((N, D), jnp.float32),
          pltpu.SemaphoreType.DMA,
      ],
  )
  def agg(src_hbm, dst_hbm, h_hbm, zeros_nd, ones_b, aggp_hbm, deg_hbm,
          src_v, dst_v, rows_v, ones_v, agg_sh, sem):
    cid = lax.axis_index("c")
    sid = lax.axis_index("s")
    wid = sid * NC + cid
    tile_rows = pl.ds(pl.multiple_of(sid * rows_a, 8), rows_a)
    last_rows = pl.ds(N - rem, rem)

    def zero_agg_sh():
      pltpu.sync_copy(zeros_nd.at[tile_rows], agg_sh.at[tile_rows])
      if rem:
        @pl.when(sid == NS - 1)
        def _():
          pltpu.sync_copy(zeros_nd.at[last_rows], agg_sh.at[last_rows])

    def copy_out(dst_hbm_ref):
      pltpu.sync_copy(agg_sh.at[tile_rows], dst_hbm_ref.at[cid].at[tile_rows])
      if rem:
        @pl.when(sid == NS - 1)
        def _():
          pltpu.sync_copy(agg_sh.at[last_rows], dst_hbm_ref.at[cid].at[last_rows])

    zero_agg_sh()
    pltpu.sync_copy(ones_b, ones_v)
    plsc.subcore_barrier()

    def bodyA(i, _):
      base = pl.multiple_of(wid * per_worker + i * batch, 8)
      pltpu.sync_copy(src_hbm.at[pl.ds(base, batch)], src_v)
      pltpu.sync_copy(dst_hbm.at[pl.ds(base, batch)], dst_v)
      pltpu.async_copy(h_hbm.at[src_v], rows_v, sem).wait()
      pltpu.sync_copy(rows_v, agg_sh.at[dst_v], add=True)
      return ()

    lax.fori_loop(0, iters, bodyA, (), unroll=False)
    plsc.subcore_barrier()
    copy_out(aggp_hbm)
    plsc.subcore_barrier()

    zero_agg_sh()
    plsc.subcore_barrier()

    def bodyB(i, _):
      base = pl.multiple_of(wid * per_worker + i * batch, 8)
      pltpu.sync_copy(dst_hbm.at[pl.ds(base, batch)], dst_v)
      pltpu.sync_copy(ones_v, agg_sh.at[dst_v], add=True)
      return ()

    lax.fori_loop(0, iters, bodyB, (), unroll=False)
    plsc.subcore_barrier()
    copy_out(deg_hbm)

  return agg


def _make_dense(N, D, H):
  blk = 400
  assert N % blk == 0
  grid = (N // blk,)

  def dense_body(h_ref, aggp_ref, deg_ref, ws_ref, wn_ref, lw_ref, lb_ref,
                 out_ref):
    aggsum = aggp_ref[0] + aggp_ref[1]
    deg = deg_ref[0][:, 0:1] + deg_ref[1][:, 0:1]
    agg = aggsum / jnp.clip(deg, 1.0)
    f32 = jnp.float32
    hid = jnp.maximum(
        jnp.dot(h_ref[...], ws_ref[...], preferred_element_type=f32)
        + jnp.dot(agg, wn_ref[...], preferred_element_type=f32), 0.0)
    out_ref[...] = (jnp.dot(hid, lw_ref[...], preferred_element_type=f32)
                    + lb_ref[...])

  return pl.pallas_call(
      dense_body,
      grid=grid,
      in_specs=[
          pl.BlockSpec((blk, D), lambda i: (i, 0)),
          pl.BlockSpec((NC, blk, D), lambda i: (0, i, 0)),
          pl.BlockSpec((NC, blk, D), lambda i: (0, i, 0)),
          pl.BlockSpec((D, H), lambda i: (0, 0)),
          pl.BlockSpec((D, H), lambda i: (0, 0)),
          pl.BlockSpec((H, H), lambda i: (0, 0)),
          pl.BlockSpec((1, H), lambda i: (0, 0)),
      ],
      out_specs=pl.BlockSpec((blk, H), lambda i: (i, 0)),
      out_shape=jax.ShapeDtypeStruct((N, H), jnp.float32),
  )


def kernel(x, edge_index, emb_table, W_self, W_neigh, lin_W, lin_b):
  N, L = x.shape
  V, D = emb_table.shape
  H = W_self.shape[1]
  E = edge_index.shape[1]

  x_flat = x.reshape(-1)
  src = edge_index[0]
  dst = edge_index[1]
  zeros_nd = jnp.zeros((N, D), jnp.float32)
  ones_b = jnp.ones((80, D), jnp.float32)

  h = _make_embed(N, L, V, D)(x_flat, emb_table)
  aggp, deg = _make_agg(N, E, D)(src, dst, h, zeros_nd, ones_b)
  return _make_dense(N, D, H)(h, aggp, deg, W_self, W_neigh, lin_W,
                              lin_b.reshape(1, H))


# speedup vs baseline: 5.7981x; 1.4148x over previous
import functools

import jax
import jax.numpy as jnp
from jax import lax
from jax.experimental import pallas as pl
from jax.experimental.pallas import tpu as pltpu
from jax.experimental.pallas import tpu_sc as plsc

NC = 2
NS = 16
NW = NC * NS


def _make_embed(N, L, V, D):
  assert L == 16 and D == 128
  npb = 8
  nb_total = N // npb
  assert N % npb == 0
  iters = (nb_total + NW - 1) // NW

  mesh = plsc.VectorSubcoreMesh(core_axis_name="c", subcore_axis_name="s",
                                num_cores=NC, num_subcores=NS)

  @functools.partial(
      pl.kernel, mesh=mesh,
      out_type=jax.ShapeDtypeStruct((N, D), jnp.float32),
      scratch_types=[
          pltpu.VMEM((npb * L,), jnp.int32),
          pltpu.VMEM((npb * L,), jnp.int32),
          pltpu.VMEM((npb * L, D), jnp.float32),
          pltpu.VMEM((npb * L, D), jnp.float32),
          pltpu.VMEM((npb, D), jnp.float32),
          pltpu.SemaphoreType.DMA,
          pltpu.SemaphoreType.DMA,
      ],
  )
  def embed(x_hbm, table_hbm, h_hbm, idx0, idx1, rows0, rows1, hbuf,
            sem0, sem1):
    wid = lax.axis_index("s") * NC + lax.axis_index("c")
    idx = (idx0, idx1)
    rows = (rows0, rows1)
    sem = (sem0, sem1)

    def batch_of(i):
      return jnp.minimum(i * NW + wid, nb_total - 1)

    def load_and_start(i, s):
      b = batch_of(i)
      pltpu.sync_copy(
          x_hbm.at[pl.ds(pl.multiple_of(b * npb * L, 8), npb * L)], idx[s])
      pltpu.async_copy(table_hbm.at[idx[s]], rows[s], sem[s])

    def consume(i, s):
      pltpu.make_async_copy(table_hbm.at[idx[s]], rows[s], sem[s]).wait()
      rv = rows[s]
      for n in range(npb):
        for c in range(D // 16):
          cols = pl.ds(c * 16, 16)
          vals = [rv[n * L + t, cols] for t in range(L)]
          while len(vals) > 1:
            vals = [vals[2 * k] + vals[2 * k + 1]
                    for k in range(len(vals) // 2)] + vals[len(vals) & ~1:]
          hbuf[n, cols] = jnp.maximum(vals[0] * (1.0 / L), 0.0)
      b = batch_of(i)
      pltpu.sync_copy(hbuf, h_hbm.at[pl.ds(pl.multiple_of(b * npb, 8), npb)])

    load_and_start(0, 0)

    def body(i, _):
      for s in (0, 1):
        @pl.when(i % 2 == s)
        def _():
          @pl.when(i + 1 < iters)
          def _():
            load_and_start(i + 1, 1 - s)
          consume(i, s)
      return ()

    lax.fori_loop(0, iters, body, (), unroll=False)

  return embed


def _make_agg(N, E, D):
  assert D == 128 and N % NS == 0
  batch = 80
  per_worker = E // NW
  assert E % NW == 0 and per_worker % batch == 0
  iters = per_worker // batch
  rows_a = (N // NS) & ~7
  rem = N - NS * rows_a
  assert rem % 8 == 0

  mesh = plsc.VectorSubcoreMesh(core_axis_name="c", subcore_axis_name="s",
                                num_cores=NC, num_subcores=NS)

  @functools.partial(
      pl.kernel, mesh=mesh,
      out_type=[jax.ShapeDtypeStruct((NC, N, D), jnp.float32),
                jax.ShapeDtypeStruct((NC, N, D), jnp.float32)],
      scratch_types=[
          pltpu.VMEM((batch,), jnp.int32),
          pltpu.VMEM((batch,), jnp.int32),
          pltpu.VMEM((batch,), jnp.int32),
          pltpu.VMEM((batch,), jnp.int32),
          pltpu.VMEM((batch, D), jnp.float32),
          pltpu.VMEM((batch, D), jnp.float32),
          pltpu.VMEM((batch, D), jnp.float32),
          pltpu.VMEM_SHARED((N, D), jnp.float32),
          pltpu.SemaphoreType.DMA,
          pltpu.SemaphoreType.DMA,
          pltpu.SemaphoreType.DMA,
      ],
  )
  def agg(src_hbm, dst_hbm, h_hbm, zeros_nd, ones_b, aggp_hbm, deg_hbm,
          src0, src1, dst0, dst1, rows0, rows1, ones_v, agg_sh,
          sem0, sem1, semd):
    cid = lax.axis_index("c")
    sid = lax.axis_index("s")
    wid = sid * NC + cid
    tile_rows = pl.ds(pl.multiple_of(sid * rows_a, 8), rows_a)
    last_rows = pl.ds(N - rem, rem)
    srcv = (src0, src1)
    dstv = (dst0, dst1)
    rows = (rows0, rows1)
    sem = (sem0, sem1)

    def zero_agg_sh():
      pltpu.sync_copy(zeros_nd.at[tile_rows], agg_sh.at[tile_rows])
      if rem:
        @pl.when(sid == NS - 1)
        def _():
          pltpu.sync_copy(zeros_nd.at[last_rows], agg_sh.at[last_rows])

    def copy_out(dst_ref):
      pltpu.sync_copy(agg_sh.at[tile_rows], dst_ref.at[cid].at[tile_rows])
      if rem:
        @pl.when(sid == NS - 1)
        def _():
          pltpu.sync_copy(agg_sh.at[last_rows], dst_ref.at[cid].at[last_rows])

    def edge_base(i):
      return pl.multiple_of(wid * per_worker + i * batch, 8)

    zero_agg_sh()
    pltpu.sync_copy(ones_b, ones_v)
    plsc.subcore_barrier()

    def load_and_start(i, s):
      base = edge_base(i)
      pltpu.sync_copy(src_hbm.at[pl.ds(base, batch)], srcv[s])
      pltpu.sync_copy(dst_hbm.at[pl.ds(base, batch)], dstv[s])
      pltpu.async_copy(h_hbm.at[srcv[s]], rows[s], sem[s])

    load_and_start(0, 0)

    def bodyA(i, _):
      for s in (0, 1):
        @pl.when(i % 2 == s)
        def _():
          @pl.when(i + 1 < iters)
          def _():
            load_and_start(i + 1, 1 - s)
          pltpu.make_async_copy(h_hbm.at[srcv[s]], rows[s], sem[s]).wait()
          pltpu.sync_copy(rows[s], agg_sh.at[dstv[s]], add=True)
      return ()

    lax.fori_loop(0, iters, bodyA, (), unroll=False)
    plsc.subcore_barrier()
    copy_out(aggp_hbm)
    plsc.subcore_barrier()

    zero_agg_sh()
    plsc.subcore_barrier()

    def load_dst(i, s):
      pltpu.async_copy(dst_hbm.at[pl.ds(edge_base(i), batch)], dstv[s], semd)

    load_dst(0, 0)

    def bodyB(i, _):
      for s in (0, 1):
        @pl.when(i % 2 == s)
        def _():
          pltpu.make_async_copy(
              dst_hbm.at[pl.ds(edge_base(i), batch)], dstv[s], semd).wait()
          @pl.when(i + 1 < iters)
          def _():
            load_dst(i + 1, 1 - s)
          pltpu.sync_copy(ones_v, agg_sh.at[dstv[s]], add=True)
      return ()

    lax.fori_loop(0, iters, bodyB, (), unroll=False)
    plsc.subcore_barrier()
    copy_out(deg_hbm)

  return agg


def _make_dense(N, D, H):
  blk = 400
  assert N % blk == 0
  grid = (N // blk,)

  def dense_body(h_ref, aggp_ref, deg_ref, ws_ref, wn_ref, lw_ref, lb_ref,
                 out_ref):
    aggsum = aggp_ref[0] + aggp_ref[1]
    deg = deg_ref[0][:, 0:1] + deg_ref[1][:, 0:1]
    agg = aggsum / jnp.clip(deg, 1.0)
    f32 = jnp.float32
    hid = jnp.maximum(
        jnp.dot(h_ref[...], ws_ref[...], preferred_element_type=f32)
        + jnp.dot(agg, wn_ref[...], preferred_element_type=f32), 0.0)
    out_ref[...] = (jnp.dot(hid, lw_ref[...], preferred_element_type=f32)
                    + lb_ref[...])

  return pl.pallas_call(
      dense_body,
      grid=grid,
      in_specs=[
          pl.BlockSpec((blk, D), lambda i: (i, 0)),
          pl.BlockSpec((NC, blk, D), lambda i: (0, i, 0)),
          pl.BlockSpec((NC, blk, D), lambda i: (0, i, 0)),
          pl.BlockSpec((D, H), lambda i: (0, 0)),
          pl.BlockSpec((D, H), lambda i: (0, 0)),
          pl.BlockSpec((H, H), lambda i: (0, 0)),
          pl.BlockSpec((1, H), lambda i: (0, 0)),
      ],
      out_specs=pl.BlockSpec((blk, H), lambda i: (i, 0)),
      out_shape=jax.ShapeDtypeStruct((N, H), jnp.float32),
  )


def kernel(x, edge_index, emb_table, W_self, W_neigh, lin_W, lin_b):
  N, L = x.shape
  V, D = emb_table.shape
  H = W_self.shape[1]
  E = edge_index.shape[1]

  x_flat = x.reshape(-1)
  src = edge_index[0]
  dst = edge_index[1]
  zeros_nd = jnp.zeros((N, D), jnp.float32)
  ones_b = jnp.ones((80, D), jnp.float32)

  h = _make_embed(N, L, V, D)(x_flat, emb_table)
  aggp, deg = _make_agg(N, E, D)(src, dst, h, zeros_nd, ones_b)
  return _make_dense(N, D, H)(h, aggp, deg, W_self, W_neigh, lin_W,
                              lin_b.reshape(1, H))


# speedup vs baseline: 6.2700x; 1.0814x over previous
import functools

import jax
import jax.numpy as jnp
from jax import lax
from jax.experimental import pallas as pl
from jax.experimental.pallas import tpu as pltpu
from jax.experimental.pallas import tpu_sc as plsc

NC = 2
NS = 16
NW = NC * NS


def _make_embed(N, L, V, D):
  assert L == 16 and D == 128
  npb = 8
  nb_total = N // npb
  assert N % npb == 0
  iters = (nb_total + NW - 1) // NW

  mesh = plsc.VectorSubcoreMesh(core_axis_name="c", subcore_axis_name="s",
                                num_cores=NC, num_subcores=NS)

  @functools.partial(
      pl.kernel, mesh=mesh,
      out_type=jax.ShapeDtypeStruct((N, D), jnp.float32),
      scratch_types=[
          pltpu.VMEM((npb * L,), jnp.int32),
          pltpu.VMEM((npb * L,), jnp.int32),
          pltpu.VMEM((npb * L, D), jnp.float32),
          pltpu.VMEM((npb * L, D), jnp.float32),
          pltpu.VMEM((npb, D), jnp.float32),
          pltpu.SemaphoreType.DMA,
          pltpu.SemaphoreType.DMA,
      ],
  )
  def embed(x_hbm, table_hbm, h_hbm, idx0, idx1, rows0, rows1, hbuf,
            sem0, sem1):
    wid = lax.axis_index("s") * NC + lax.axis_index("c")
    idx = (idx0, idx1)
    rows = (rows0, rows1)
    sem = (sem0, sem1)

    def batch_of(i):
      return jnp.minimum(i * NW + wid, nb_total - 1)

    def load_and_start(i, s):
      b = batch_of(i)
      pltpu.sync_copy(
          x_hbm.at[pl.ds(pl.multiple_of(b * npb * L, 8), npb * L)], idx[s])
      pltpu.async_copy(table_hbm.at[idx[s]], rows[s], sem[s])

    def consume(i, s):
      pltpu.make_async_copy(table_hbm.at[idx[s]], rows[s], sem[s]).wait()
      rv = rows[s]
      for n in range(npb):
        for c in range(D // 16):
          cols = pl.ds(c * 16, 16)
          vals = [rv[n * L + t, cols] for t in range(L)]
          while len(vals) > 1:
            vals = [vals[2 * k] + vals[2 * k + 1]
                    for k in range(len(vals) // 2)] + vals[len(vals) & ~1:]
          hbuf[n, cols] = jnp.maximum(vals[0] * (1.0 / L), 0.0)
      b = batch_of(i)
      pltpu.sync_copy(hbuf, h_hbm.at[pl.ds(pl.multiple_of(b * npb, 8), npb)])

    load_and_start(0, 0)

    def body(i, _):
      for s in (0, 1):
        @pl.when(i % 2 == s)
        def _():
          @pl.when(i + 1 < iters)
          def _():
            load_and_start(i + 1, 1 - s)
          consume(i, s)
      return ()

    lax.fori_loop(0, iters, body, (), unroll=False)

  return embed


def _make_agg(N, E, D):
  assert D == 128 and N % NS == 0
  batch = 128
  per_worker = E // NW
  assert E % NW == 0
  iters = per_worker // batch
  tail = per_worker - iters * batch
  assert tail % 8 == 0
  rows_a = (N // NS) & ~7
  rem = N - NS * rows_a
  assert rem % 8 == 0

  mesh = plsc.VectorSubcoreMesh(core_axis_name="c", subcore_axis_name="s",
                                num_cores=NC, num_subcores=NS)

  @functools.partial(
      pl.kernel, mesh=mesh,
      out_type=[jax.ShapeDtypeStruct((NC, N, D), jnp.float32),
                jax.ShapeDtypeStruct((NC, N, D), jnp.float32)],
      scratch_types=[
          pltpu.VMEM((batch,), jnp.int32),
          pltpu.VMEM((batch,), jnp.int32),
          pltpu.VMEM((batch,), jnp.int32),
          pltpu.VMEM((batch,), jnp.int32),
          pltpu.VMEM((tail,), jnp.int32),
          pltpu.VMEM((tail,), jnp.int32),
          pltpu.VMEM((batch, D), jnp.float32),
          pltpu.VMEM((batch, D), jnp.float32),
          pltpu.VMEM((tail, D), jnp.float32),
          pltpu.VMEM_SHARED((N, D), jnp.float32),
          pltpu.SemaphoreType.DMA,
          pltpu.SemaphoreType.DMA,
          pltpu.SemaphoreType.DMA,
      ],
  )
  def agg(src_hbm, dst_hbm, h_hbm, zeros_nd, ones_b, aggp_hbm, deg_hbm,
          src0, src1, dst0, dst1, srcT, dstT, rows0, rows1, rowsT,
          agg_sh, sem0, sem1, semd):
    cid = lax.axis_index("c")
    sid = lax.axis_index("s")
    wid = sid * NC + cid
    tile_rows = pl.ds(pl.multiple_of(sid * rows_a, 8), rows_a)
    last_rows = pl.ds(N - rem, rem)
    srcv = (src0, src1)
    dstv = (dst0, dst1)
    rows = (rows0, rows1)
    sem = (sem0, sem1)

    def zero_agg_sh():
      pltpu.sync_copy(zeros_nd.at[tile_rows], agg_sh.at[tile_rows])
      if rem:
        @pl.when(sid == NS - 1)
        def _():
          pltpu.sync_copy(zeros_nd.at[last_rows], agg_sh.at[last_rows])

    def copy_out(dst_ref):
      pltpu.sync_copy(agg_sh.at[tile_rows], dst_ref.at[cid].at[tile_rows])
      if rem:
        @pl.when(sid == NS - 1)
        def _():
          pltpu.sync_copy(agg_sh.at[last_rows], dst_ref.at[cid].at[last_rows])

    def edge_base(i):
      return pl.multiple_of(wid * per_worker + i * batch, 8)

    tail_base = pl.multiple_of(wid * per_worker + iters * batch, 8)

    zero_agg_sh()
    plsc.subcore_barrier()

    def load_and_start(i, s):
      base = edge_base(i)
      pltpu.sync_copy(src_hbm.at[pl.ds(base, batch)], srcv[s])
      pltpu.sync_copy(dst_hbm.at[pl.ds(base, batch)], dstv[s])
      pltpu.async_copy(h_hbm.at[srcv[s]], rows[s], sem[s])

    load_and_start(0, 0)

    def bodyA(i, _):
      for s in (0, 1):
        @pl.when(i % 2 == s)
        def _():
          @pl.when(i + 1 < iters)
          def _():
            load_and_start(i + 1, 1 - s)
          pltpu.make_async_copy(h_hbm.at[srcv[s]], rows[s], sem[s]).wait()
          pltpu.sync_copy(rows[s], agg_sh.at[dstv[s]], add=True)
      return ()

    lax.fori_loop(0, iters, bodyA, (), unroll=False)
    if tail:
      pltpu.sync_copy(src_hbm.at[pl.ds(tail_base, tail)], srcT)
      pltpu.sync_copy(dst_hbm.at[pl.ds(tail_base, tail)], dstT)
      pltpu.async_copy(h_hbm.at[srcT], rowsT, sem0).wait()
      pltpu.sync_copy(rowsT, agg_sh.at[dstT], add=True)
    plsc.subcore_barrier()
    copy_out(aggp_hbm)
    plsc.subcore_barrier()

    zero_agg_sh()
    pltpu.sync_copy(ones_b, rows0)
    plsc.subcore_barrier()

    def load_dst(i, s):
      pltpu.async_copy(dst_hbm.at[pl.ds(edge_base(i), batch)], dstv[s], semd)

    load_dst(0, 0)

    def bodyB(i, _):
      for s in (0, 1):
        @pl.when(i % 2 == s)
        def _():
          pltpu.make_async_copy(
              dst_hbm.at[pl.ds(edge_base(i), batch)], dstv[s], semd).wait()
          @pl.when(i + 1 < iters)
          def _():
            load_dst(i + 1, 1 - s)
          pltpu.sync_copy(rows0, agg_sh.at[dstv[s]], add=True)
      return ()

    lax.fori_loop(0, iters, bodyB, (), unroll=False)
    if tail:
      pltpu.sync_copy(dst_hbm.at[pl.ds(tail_base, tail)], dstT)
      pltpu.sync_copy(rows0.at[pl.ds(0, tail)], agg_sh.at[dstT], add=True)
    plsc.subcore_barrier()
    copy_out(deg_hbm)

  return agg


def _make_dense(N, D, H):
  blk = 400
  assert N % blk == 0
  grid = (N // blk,)

  def dense_body(h_ref, aggp_ref, deg_ref, ws_ref, wn_ref, lw_ref, lb_ref,
                 out_ref):
    aggsum = aggp_ref[0] + aggp_ref[1]
    deg = deg_ref[0][:, 0:1] + deg_ref[1][:, 0:1]
    agg = aggsum / jnp.clip(deg, 1.0)
    f32 = jnp.float32
    hid = jnp.maximum(
        jnp.dot(h_ref[...], ws_ref[...], preferred_element_type=f32)
        + jnp.dot(agg, wn_ref[...], preferred_element_type=f32), 0.0)
    out_ref[...] = (jnp.dot(hid, lw_ref[...], preferred_element_type=f32)
                    + lb_ref[...])

  return pl.pallas_call(
      dense_body,
      grid=grid,
      in_specs=[
          pl.BlockSpec((blk, D), lambda i: (i, 0)),
          pl.BlockSpec((NC, blk, D), lambda i: (0, i, 0)),
          pl.BlockSpec((NC, blk, D), lambda i: (0, i, 0)),
          pl.BlockSpec((D, H), lambda i: (0, 0)),
          pl.BlockSpec((D, H), lambda i: (0, 0)),
          pl.BlockSpec((H, H), lambda i: (0, 0)),
          pl.BlockSpec((1, H), lambda i: (0, 0)),
      ],
      out_specs=pl.BlockSpec((blk, H), lambda i: (i, 0)),
      out_shape=jax.ShapeDtypeStruct((N, H), jnp.float32),
  )


def kernel(x, edge_index, emb_table, W_self, W_neigh, lin_W, lin_b):
  N, L = x.shape
  V, D = emb_table.shape
  H = W_self.shape[1]
  E = edge_index.shape[1]

  x_flat = x.reshape(-1)
  src = edge_index[0]
  dst = edge_index[1]
  zeros_nd = jnp.zeros((N, D), jnp.float32)
  ones_b = jnp.ones((128, D), jnp.float32)

  h = _make_embed(N, L, V, D)(x_flat, emb_table)
  aggp, deg = _make_agg(N, E, D)(src, dst, h, zeros_nd, ones_b)
  return _make_dense(N, D, H)(h, aggp, deg, W_self, W_neigh, lin_W,
                              lin_b.reshape(1, H))
